# baseline (device time: 235292 ns/iter reference)
import jax
import jax.numpy as jnp
from jax import lax
from jax.experimental import pallas as pl
from jax.experimental.pallas import tpu as pltpu

NB = 2
NC = 8
CPB = NC // (NB // 2)


def kernel(x, W):
    t, d = x.shape
    _, v_loc = W.shape
    v_glob = 2 * v_loc
    brows = t // NB
    half_rows = t // 2
    rc = half_rows // NC

    def body(
        x_ref, w_ref, out_ref, logits, w_tiles, stat_tx, stat_rx,
        copy_sem, wsems, stat_sems, sems,
    ):
        mx = lax.axis_index("x")
        my = lax.axis_index("y")
        nbr = (1 - mx, my)
        nbr_y = (mx, 1 - my)
        r0 = my * half_rows
        q0 = (1 - my) * half_rows
        own_c0 = mx * v_loc
        oth_c0 = (1 - mx) * v_loc

        barrier = pltpu.get_barrier_semaphore()
        for nb in (nbr, nbr_y):
            pl.semaphore_signal(
                barrier, inc=1, device_id=nb,
                device_id_type=pl.DeviceIdType.MESH,
            )
        pl.semaphore_wait(barrier, 2)

        def run_blocks(my_s):
            r0s = my_s * half_rows
            for k in range(NB):
                rofs = ((my_s * (NB // 2) + k) % NB) * brows
                rs = slice(rofs, rofs + brows)
                x_blk = x_ref[rs, :]
                m_b = jnp.full((brows, 1), -1e30, jnp.float32)
                wtile = v_loc // 8

                def wdma(j, slot):
                    return pltpu.make_async_copy(
                        w_ref.at[:, pl.ds(j * wtile, wtile)],
                        w_tiles.at[slot],
                        wsems.at[slot],
                    )

                wdma(0, 0).start()
                for j in range(8):
                    slot = j % 2
                    if j + 1 < 8:
                        wdma(j + 1, 1 - slot).start()
                    wdma(j, slot).wait()
                    cs = slice(j * wtile, (j + 1) * wtile)
                    tl = jnp.dot(
                        x_blk, w_tiles[slot],
                        preferred_element_type=jnp.float32,
                    )
                    logits[rs, cs] = tl
                    m_b = jnp.maximum(
                        m_b, jnp.max(tl, axis=1, keepdims=True)
                    )
                s_b = jnp.zeros((brows, 1), jnp.float32)
                for j in range(8):
                    cs = slice(j * wtile, (j + 1) * wtile)
                    e_t = jnp.exp(logits[rs, cs] - m_b)
                    logits[rs, cs] = e_t
                    s_b = s_b + jnp.sum(e_t, axis=1, keepdims=True)
                stat_tx[rs, 0:128] = jnp.broadcast_to(m_b, (brows, 128))
                stat_tx[rs, 128:256] = jnp.broadcast_to(s_b, (brows, 128))
                pltpu.make_async_remote_copy(
                    src_ref=stat_tx.at[rs, :],
                    dst_ref=stat_rx.at[rs, :],
                    send_sem=stat_sems.at[0, k],
                    recv_sem=stat_sems.at[1, k],
                    device_id=nbr,
                    device_id_type=pl.DeviceIdType.MESH,
                ).start()

                pltpu.make_async_remote_copy(
                    src_ref=stat_tx.at[rs, :],
                    dst_ref=stat_rx.at[rs, :],
                    send_sem=stat_sems.at[0, k],
                    recv_sem=stat_sems.at[1, k],
                    device_id=nbr,
                    device_id_type=pl.DeviceIdType.MESH,
                ).wait_recv()
                m_rem = stat_rx[rs, 0:1]
                s_rem = stat_rx[rs, 128:129]
                m_glob = jnp.maximum(m_b, m_rem)
                s_glob = s_b * jnp.exp(m_b - m_glob) + s_rem * jnp.exp(
                    m_rem - m_glob
                )
                scale = jnp.exp(m_b - m_glob) / s_glob
                logits[rs, :] = logits[rs, :] * scale
                if k < NB // 2:
                    for c in range(k * CPB, (k + 1) * CPB):
                        cs = slice(r0s + c * rc, r0s + (c + 1) * rc)
                        pltpu.make_async_remote_copy(
                            src_ref=logits.at[cs, :],
                            dst_ref=out_ref.at[cs, pl.ds(own_c0, v_loc)],
                            send_sem=sems.at[0, c],
                            recv_sem=sems.at[1, c],
                            device_id=nbr,
                            device_id_type=pl.DeviceIdType.MESH,
                        ).start()

        @pl.when(my == 0)
        def _():
            run_blocks(0)

        @pl.when(my == 1)
        def _():
            run_blocks(1)

        local_cp = pltpu.make_async_copy(
            logits, out_ref.at[:, pl.ds(own_c0, v_loc)], copy_sem
        )
        local_cp.start()

        y_out = []
        for c in range(NC):
            rs = pl.ds(r0 + c * rc, rc)
            x_in = pltpu.make_async_remote_copy(
                src_ref=logits.at[rs, :],
                dst_ref=out_ref.at[rs, pl.ds(oth_c0, v_loc)],
                send_sem=sems.at[0, c],
                recv_sem=sems.at[1, c],
                device_id=nbr,
                device_id_type=pl.DeviceIdType.MESH,
            )
            x_in.wait_recv()
            fwd = pltpu.make_async_remote_copy(
                src_ref=out_ref.at[rs, pl.ds(oth_c0, v_loc)],
                dst_ref=out_ref.at[rs, pl.ds(oth_c0, v_loc)],
                send_sem=sems.at[2, c],
                recv_sem=sems.at[3, c],
                device_id=nbr_y,
                device_id_type=pl.DeviceIdType.MESH,
            )
            fwd.start()
            y_out.append(fwd)

        for c in range(NC):
            rs = pl.ds(q0 + c * rc, rc)
            y_in = pltpu.make_async_remote_copy(
                src_ref=out_ref.at[rs, pl.ds(oth_c0, v_loc)],
                dst_ref=out_ref.at[rs, pl.ds(oth_c0, v_loc)],
                send_sem=sems.at[2, c],
                recv_sem=sems.at[3, c],
                device_id=nbr_y,
                device_id_type=pl.DeviceIdType.MESH,
            )
            y_in.wait_recv()
        for k in range(NB):
            pltpu.make_async_remote_copy(
                src_ref=stat_tx.at[0:brows, :],
                dst_ref=stat_rx.at[0:brows, :],
                send_sem=stat_sems.at[0, k],
                recv_sem=stat_sems.at[1, k],
                device_id=nbr,
                device_id_type=pl.DeviceIdType.MESH,
            ).wait_send()
        for c in range(NC):
            pltpu.make_async_remote_copy(
                src_ref=logits.at[0:rc, :],
                dst_ref=out_ref.at[0:rc, pl.ds(own_c0, v_loc)],
                send_sem=sems.at[0, c],
                recv_sem=sems.at[1, c],
                device_id=nbr,
                device_id_type=pl.DeviceIdType.MESH,
            ).wait_send()
        for rd in y_out:
            rd.wait_send()
        local_cp.wait()

    return pl.pallas_call(
        body,
        out_shape=jax.ShapeDtypeStruct((t, v_glob), jnp.float32),
        in_specs=[
            pl.BlockSpec(memory_space=pltpu.VMEM),
            pl.BlockSpec(memory_space=pl.ANY),
        ],
        out_specs=pl.BlockSpec(memory_space=pl.ANY),
        scratch_shapes=[
            pltpu.VMEM((t, v_loc), jnp.float32),
            pltpu.VMEM((2, d, v_loc // 8), jnp.float32),
            pltpu.VMEM((t, 256), jnp.float32),
            pltpu.VMEM((t, 256), jnp.float32),
            pltpu.SemaphoreType.DMA,
            pltpu.SemaphoreType.DMA((2,)),
            pltpu.SemaphoreType.DMA((2, NB)),
            pltpu.SemaphoreType.DMA((4, NC)),
        ],
        compiler_params=pltpu.CompilerParams(
            collective_id=0,
            vmem_limit_bytes=62 * 1024 * 1024,
        ),
    )(x, W)


# device time: 159962 ns/iter; 1.4709x vs baseline; 1.4709x over previous
import jax
import jax.numpy as jnp
from jax import lax
from jax.experimental import pallas as pl
from jax.experimental.pallas import tpu as pltpu

N_TILES = 8
NC = 8


def kernel(x, W):
    t, d = x.shape
    _, v_loc = W.shape
    v_glob = 2 * v_loc
    tile = v_loc // N_TILES

    def body(
        x_ref, w_ref, out_ref, w_tiles, logits, stat_tx, stat_rx,
        wsems, copy_sem, stat_sems, sems,
    ):
        mx = lax.axis_index("x")
        my = lax.axis_index("y")
        nbr = (1 - mx, my)
        nbr_y = (mx, 1 - my)

        barrier = pltpu.get_barrier_semaphore()
        for nb in (nbr, nbr_y):
            pl.semaphore_signal(
                barrier, inc=1, device_id=nb,
                device_id_type=pl.DeviceIdType.MESH,
            )
        pl.semaphore_wait(barrier, 2)

        xv = x_ref[...]

        def wdma(i, slot):
            return pltpu.make_async_copy(
                w_ref.at[:, pl.ds(i * tile, tile)],
                w_tiles.at[slot],
                wsems.at[slot],
            )

        wdma(0, 0).start()
        m_loc = jnp.full((t, 1), -1e30, jnp.float32)
        for i in range(N_TILES):
            slot = i % 2
            if i + 1 < N_TILES:
                wdma(i + 1, 1 - slot).start()
            wdma(i, slot).wait()
            tl = jnp.dot(xv, w_tiles[slot], preferred_element_type=jnp.float32)
            logits[:, i * tile : (i + 1) * tile] = tl
            m_loc = jnp.maximum(m_loc, jnp.max(tl, axis=1, keepdims=True))

        s_loc = jnp.zeros((t, 1), jnp.float32)
        for i in range(N_TILES):
            sl = slice(i * tile, (i + 1) * tile)
            e_t = jnp.exp(logits[:, sl] - m_loc)
            logits[:, sl] = e_t
            s_loc = s_loc + jnp.sum(e_t, axis=1, keepdims=True)

        stat_tx[:, 0:128] = jnp.broadcast_to(m_loc, (t, 128))
        stat_tx[:, 128:256] = jnp.broadcast_to(s_loc, (t, 128))
        stat_rdma = pltpu.make_async_remote_copy(
            src_ref=stat_tx,
            dst_ref=stat_rx,
            send_sem=stat_sems.at[0],
            recv_sem=stat_sems.at[1],
            device_id=nbr,
            device_id_type=pl.DeviceIdType.MESH,
        )
        stat_rdma.start()
        stat_rdma.wait()

        m_rem = stat_rx[:, 0:1]
        s_rem = stat_rx[:, 128:129]
        m_glob = jnp.maximum(m_loc, m_rem)
        s_glob = s_loc * jnp.exp(m_loc - m_glob) + s_rem * jnp.exp(
            m_rem - m_glob
        )
        scale = jnp.exp(m_loc - m_glob) / s_glob

        for i in range(N_TILES):
            sl = slice(i * tile, (i + 1) * tile)
            logits[:, sl] = logits[:, sl] * scale

        half_rows = t // 2
        rc = half_rows // NC
        r0 = my * half_rows
        q0 = (1 - my) * half_rows
        own_c0 = mx * v_loc
        oth_c0 = (1 - mx) * v_loc

        local_cp = pltpu.make_async_copy(
            logits, out_ref.at[:, pl.ds(own_c0, v_loc)], copy_sem
        )
        local_cp.start()

        x_out = []
        for c in range(NC):
            rs = pl.ds(r0 + c * rc, rc)
            rd = pltpu.make_async_remote_copy(
                src_ref=logits.at[rs, :],
                dst_ref=out_ref.at[rs, pl.ds(own_c0, v_loc)],
                send_sem=sems.at[0, c],
                recv_sem=sems.at[1, c],
                device_id=nbr,
                device_id_type=pl.DeviceIdType.MESH,
            )
            rd.start()
            x_out.append(rd)

        y_out = []
        for c in range(NC):
            rs = pl.ds(r0 + c * rc, rc)
            x_in = pltpu.make_async_remote_copy(
                src_ref=logits.at[rs, :],
                dst_ref=out_ref.at[rs, pl.ds(oth_c0, v_loc)],
                send_sem=sems.at[0, c],
                recv_sem=sems.at[1, c],
                device_id=nbr,
                device_id_type=pl.DeviceIdType.MESH,
            )
            x_in.wait_recv()
            fwd = pltpu.make_async_remote_copy(
                src_ref=out_ref.at[rs, pl.ds(oth_c0, v_loc)],
                dst_ref=out_ref.at[rs, pl.ds(oth_c0, v_loc)],
                send_sem=sems.at[2, c],
                recv_sem=sems.at[3, c],
                device_id=nbr_y,
                device_id_type=pl.DeviceIdType.MESH,
            )
            fwd.start()
            y_out.append(fwd)

        for c in range(NC):
            rs = pl.ds(q0 + c * rc, rc)
            y_in = pltpu.make_async_remote_copy(
                src_ref=out_ref.at[rs, pl.ds(oth_c0, v_loc)],
                dst_ref=out_ref.at[rs, pl.ds(oth_c0, v_loc)],
                send_sem=sems.at[2, c],
                recv_sem=sems.at[3, c],
                device_id=nbr_y,
                device_id_type=pl.DeviceIdType.MESH,
            )
            y_in.wait_recv()
        for rd in x_out:
            rd.wait_send()
        for rd in y_out:
            rd.wait_send()
        local_cp.wait()

    return pl.pallas_call(
        body,
        out_shape=jax.ShapeDtypeStruct((t, v_glob), jnp.float32),
        in_specs=[
            pl.BlockSpec(memory_space=pltpu.VMEM),
            pl.BlockSpec(memory_space=pl.ANY),
        ],
        out_specs=pl.BlockSpec(memory_space=pl.ANY),
        scratch_shapes=[
            pltpu.VMEM((2, d, tile), jnp.float32),
            pltpu.VMEM((t, v_loc), jnp.float32),
            pltpu.VMEM((t, 256), jnp.float32),
            pltpu.VMEM((t, 256), jnp.float32),
            pltpu.SemaphoreType.DMA((2,)),
            pltpu.SemaphoreType.DMA,
            pltpu.SemaphoreType.DMA((2,)),
            pltpu.SemaphoreType.DMA((4, NC)),
        ],
        compiler_params=pltpu.CompilerParams(collective_id=0),
    )(x, W)
